# Initial kernel scaffold; baseline (speedup 1.0000x reference)
import jax
import jax.numpy as jnp
from jax.experimental import pallas as pl


def kernel(inputs, targets):
    raise NotImplementedError("write your pallas kernel here")


# speedup vs baseline: 10271.7714x; 10271.7714x over previous
import functools

import jax
import jax.numpy as jnp
from jax.experimental import pallas as pl

_MARGIN = 0.03
_K1 = 20
_K2 = 6
_LAMBDA = 0.3


def _rerank_body(x_ref, t_ref, loss_ref, final_ref, ap_ref, an_ref):
    n = final_ref.shape[0]
    x = x_ref[:]

    x2 = jnp.sum(x * x, axis=1, keepdims=True)
    g = jax.lax.dot_general(
        x, x, (((1,), (1,)), ((), ())), preferred_element_type=jnp.float32
    )
    d = jnp.maximum(x2 + x2.T - 2.0 * g, 0.0)

    ck = 8
    rank = jnp.zeros((n, n), jnp.float32)
    ja = jax.lax.broadcasted_iota(jnp.int32, (n, ck, n), 2)
    for c in range(0, n, ck):
        dk = d[:, c : c + ck]
        ka = jax.lax.broadcasted_iota(jnp.int32, (n, ck, n), 1) + c
        less = (dk[:, :, None] < d[:, None, :]).astype(jnp.float32)
        tie = ((dk[:, :, None] == d[:, None, :]) & (ka < ja)).astype(
            jnp.float32
        )
        rank = rank + jnp.sum(less + tie, axis=1)

    t21 = jnp.where(rank < float(_K1 + 1), 1.0, 0.0)
    t11 = jnp.where(rank < float(round(_K1 / 2.0) + 1), 1.0, 0.0)
    m_f = t21 * t21.T
    mh_f = t11 * t11.T

    cnt = jnp.sum(mh_f, axis=1)
    min_req = jnp.floor(cnt * (2.0 / 3.0)) + 1.0
    inter = jax.lax.dot_general(
        m_f, mh_f, (((1,), (1,)), ((), ())), preferred_element_type=jnp.float32
    )
    cond_f = jnp.where(inter >= min_req[None, :], m_f, 0.0)
    spread = jnp.dot(cond_f, mh_f, preferred_element_type=jnp.float32)

    w = jnp.where(m_f + spread > 0.0, jnp.exp(-d), 0.0)
    v = w / jnp.sum(w, axis=1, keepdims=True)
    a6 = jnp.where(rank < float(_K2), 1.0, 0.0)
    v2 = jnp.dot(a6, v, preferred_element_type=jnp.float32) * (1.0 / _K2)

    cols = []
    for c in range(0, n, ck):
        vc = v2[c : c + ck, :]
        tm = jnp.sum(jnp.minimum(v2[:, None, :], vc[None, :, :]), axis=2)
        cols.append(tm)
    tmin = jnp.concatenate(cols, axis=1)
    jac = 1.0 - tmin / (2.0 - tmin)
    final = jac * (1.0 - _LAMBDA) + d * _LAMBDA
    final_ref[:] = final

    t = t_ref[:]
    same = t.T == t
    neg_inf = jnp.float32(-jnp.inf)
    pos_inf = jnp.float32(jnp.inf)
    ap = jnp.max(jnp.where(same, final, neg_inf), axis=1)
    an = jnp.min(jnp.where(same, pos_inf, final), axis=1)
    ap_ref[:] = ap[None, :]
    an_ref[:] = an[None, :]
    loss_ref[:] = jnp.mean(jnp.maximum(ap - an + _MARGIN, 0.0)).reshape(1, 1)


@functools.partial(jax.jit, static_argnames=("interpret",))
def _rerank(inputs, targets, interpret=False):
    n = inputs.shape[0]
    loss, final, ap, an = pl.pallas_call(
        _rerank_body,
        out_shape=[
            jax.ShapeDtypeStruct((1, 1), jnp.float32),
            jax.ShapeDtypeStruct((n, n), jnp.float32),
            jax.ShapeDtypeStruct((1, n), jnp.float32),
            jax.ShapeDtypeStruct((1, n), jnp.float32),
        ],
        interpret=interpret,
    )(inputs, targets.reshape(1, n).astype(jnp.int32))
    return loss[0, 0], final, ap[0], an[0]


def kernel(inputs, targets):
    return _rerank(inputs, targets)
